# baseline (speedup 1.0000x reference)
import functools

import jax
import jax.numpy as jnp
from jax import lax
from jax.experimental import pallas as pl
from jax.experimental.pallas import tpu as pltpu
from jax.experimental.pallas import tpu_sc as plsc

SEQ = 200
BATCH = 4096
DIM = 64
OUT_DIM = 128
VOCAB = 1000000

NC = 2
NS = 16
NW = NC * NS
BPW = BATCH // NW
NBUF = 4
LANES = 16
HC = DIM // LANES

CHUNK = 256
NFULL = VOCAB // CHUNK
TAIL = VOCAB - NFULL * CHUNK
INROWS = CHUNK + 8
APW = (NFULL + NW - 1) // NW


def _widen_body(tab_hbm, out_hbm, in0, in1, out_v, si0, si1):
    ins = (in0, in1)
    sis = (si0, si1)
    wid = lax.axis_index("s") * NC + lax.axis_index("c")

    def regcopy(nrows, in_v):
        def row(i, carry):
            nxt = tuple(in_v[i + 1, pl.ds(c * LANES, LANES)] for c in range(HC))
            for c in range(HC):
                out_v[i, pl.ds(c * LANES, LANES)] = carry[c]
                out_v[i, pl.ds(DIM + c * LANES, LANES)] = nxt[c]
            return nxt
        init = tuple(in_v[0, pl.ds(c * LANES, LANES)] for c in range(HC))
        lax.fori_loop(0, nrows, row, init)

    def fire(c, slot):
        r = pl.multiple_of(c * CHUNK, CHUNK)
        pltpu.make_async_copy(
            tab_hbm.at[pl.ds(r, INROWS), :], ins[slot], sis[slot]).start()

    fire(wid, 0)
    fire(wid + NW, 1)

    def pair(kk, carry):
        for par in range(2):
            k = kk * 2 + par
            c = wid + NW * k

            @pl.when(c < NFULL)
            def _(par=par, c=c, k=k):
                pltpu.make_async_copy(
                    tab_hbm.at[pl.ds(pl.multiple_of(c * CHUNK, CHUNK), INROWS), :],
                    ins[par], sis[par]).wait()
                regcopy(CHUNK, ins[par])
                c_next = c + 2 * NW

                @pl.when(c_next < NFULL)
                def _f(par=par, c_next=c_next):
                    fire(c_next, par)

                q = pl.multiple_of(c * CHUNK, CHUNK)
                pltpu.sync_copy(out_v, out_hbm.at[pl.ds(q, CHUNK), :])

        return carry

    lax.fori_loop(0, (APW + 1) // 2, pair, 0)

    @pl.when(wid == 0)
    def _tail():
        pltpu.sync_copy(tab_hbm.at[pl.ds(NFULL * CHUNK, TAIL), :],
                        in0.at[pl.ds(0, TAIL), :])
        regcopy(TAIL, in0)
        pltpu.sync_copy(out_v.at[pl.ds(0, TAIL), :],
                        out_hbm.at[pl.ds(NFULL * CHUNK, TAIL), :])


def _pool_body(x_hbm, tab2_hbm, out_hbm, idx_v, acc_v,
               b0, b1, b2, b3, s0, s1, s2, s3):
    bufs = (b0, b1, b2, b3)
    sems = (s0, s1, s2, s3)
    wid = lax.axis_index("s") * NC + lax.axis_index("c")
    base = pl.multiple_of(wid * BPW, BPW)

    pltpu.sync_copy(x_hbm.at[:, pl.ds(base, BPW)], idx_v)

    neg = jnp.full((LANES,), -jnp.inf, dtype=jnp.float32)

    def init_row(i, carry):
        for c in range(HC):
            acc_v[i, pl.ds(c * LANES, LANES)] = neg
        return carry

    lax.fori_loop(0, BPW, init_row, 0)

    for k in range(NBUF):
        pltpu.make_async_copy(
            tab2_hbm.at[idx_v.at[k]], bufs[k], sems[k]).start()

    def group(g, carry):
        for k in range(NBUF):
            s_cur = g * NBUF + k
            pltpu.make_async_copy(
                tab2_hbm.at[idx_v.at[s_cur]], bufs[k], sems[k]).wait()

            def row(i, c2, _buf=bufs[k]):
                for c in range(HC):
                    sl = pl.ds(c * LANES, LANES)
                    acc_v[i, sl] = jnp.maximum(acc_v[i, sl], _buf[i, sl])
                return c2

            lax.fori_loop(0, BPW, row, 0)

            s_next = s_cur + NBUF

            @pl.when(s_next < SEQ)
            def _fire(_buf=bufs[k], _sem=sems[k], _s=s_next):
                pltpu.make_async_copy(
                    tab2_hbm.at[idx_v.at[_s]], _buf, _sem).start()

        return carry

    lax.fori_loop(0, SEQ // NBUF, group, 0)

    pltpu.sync_copy(acc_v, out_hbm.at[pl.ds(base, BPW), :])


def _linear_body(p_ref, w_ref, b_ref, o_ref):
    o_ref[...] = lax.dot_general(
        p_ref[...][:, :DIM], w_ref[...], (((1,), (1,)), ((), ())),
        preferred_element_type=jnp.float32) + b_ref[...]


@jax.jit
def _run(x, table, W, b):
    mesh = plsc.VectorSubcoreMesh(core_axis_name="c", subcore_axis_name="s")
    params = pltpu.CompilerParams(use_tc_tiling_on_sc=True)
    widen = pl.kernel(
        _widen_body,
        out_type=jax.ShapeDtypeStruct((VOCAB, 2 * DIM), jnp.float32),
        mesh=mesh,
        scratch_types=[
            pltpu.VMEM((INROWS, DIM), jnp.float32),
            pltpu.VMEM((INROWS, DIM), jnp.float32),
            pltpu.VMEM((CHUNK, 2 * DIM), jnp.float32),
            pltpu.SemaphoreType.DMA,
            pltpu.SemaphoreType.DMA,
        ],
        compiler_params=params,
    )
    pool = pl.kernel(
        _pool_body,
        out_type=jax.ShapeDtypeStruct((BATCH, 2 * DIM), jnp.float32),
        mesh=mesh,
        scratch_types=[
            pltpu.VMEM((SEQ, BPW), jnp.int32),
            pltpu.VMEM((BPW, 2 * DIM), jnp.float32),
        ] + [pltpu.VMEM((BPW, 2 * DIM), jnp.float32)] * NBUF
          + [pltpu.SemaphoreType.DMA] * NBUF,
        compiler_params=params,
    )
    tab2 = widen(table)
    pooled = pool(x, tab2)
    blk = 512
    return pl.pallas_call(
        _linear_body,
        out_shape=jax.ShapeDtypeStruct((BATCH, OUT_DIM), jnp.float32),
        grid=(BATCH // blk,),
        in_specs=[
            pl.BlockSpec((blk, 2 * DIM), lambda i: (i, 0)),
            pl.BlockSpec((OUT_DIM, DIM), lambda i: (0, 0)),
            pl.BlockSpec((1, OUT_DIM), lambda i: (0, 0)),
        ],
        out_specs=pl.BlockSpec((blk, OUT_DIM), lambda i: (i, 0)),
    )(pooled, W, b.reshape(1, OUT_DIM))


def kernel(x, table, W, b):
    return _run(x.astype(jnp.int32), table, W, b)


# speedup vs baseline: 1.3849x; 1.3849x over previous
import functools

import jax
import jax.numpy as jnp
from jax import lax
from jax.experimental import pallas as pl
from jax.experimental.pallas import tpu as pltpu
from jax.experimental.pallas import tpu_sc as plsc

SEQ = 200
BATCH = 4096
DIM = 64
OUT_DIM = 128
VOCAB = 1000000

NC = 2
NS = 16
NW = NC * NS
BPW = BATCH // NW
NBUF = 3
LANES = 16
HC = DIM // LANES

TBLK = 8192
PAIR = 491520
H = VOCAB - PAIR
SPAD = 224


def _pool_body(x_hbm, tab2_hbm, out_hbm, idx_v, gidx_v, acc_v,
               b0, b1, b2, s0, s1, s2):
    bufs = (b0, b1, b2)
    sems = (s0, s1, s2)
    wid = lax.axis_index("s") * NC + lax.axis_index("c")
    base = pl.multiple_of(wid * BPW, BPW)

    pltpu.sync_copy(x_hbm.at[:, pl.ds(base, BPW)], idx_v)

    def prep(j, carry):
        sl = pl.ds((j & (BPW // LANES - 1)) * LANES, LANES)
        i = j >> 3
        v = idx_v[i, sl]
        gidx_v[i, sl] = jnp.where(v >= H, v - PAIR, v)
        return carry

    lax.fori_loop(0, SEQ * (BPW // LANES), prep, 0)

    neg = jnp.full((LANES,), -jnp.inf, dtype=jnp.float32)

    def init_row(i, carry):
        for c in range(HC):
            acc_v[i, pl.ds(c * LANES, LANES)] = neg
        return carry

    lax.fori_loop(0, BPW, init_row, 0)

    for k in range(NBUF):
        pltpu.make_async_copy(
            tab2_hbm.at[gidx_v.at[k]], bufs[k], sems[k]).start()

    def group(g, carry):
        for k in range(NBUF):
            s_cur = g * NBUF + k

            @pl.when(s_cur < SEQ)
            def _(k=k, s_cur=s_cur):
                pltpu.make_async_copy(
                    tab2_hbm.at[gidx_v.at[s_cur]], bufs[k], sems[k]).wait()

                def row16(ch, c2, _buf=bufs[k]):
                    i0 = pl.multiple_of(ch * LANES, LANES)
                    v = idx_v[s_cur, pl.ds(i0, LANES)]
                    for l in range(LANES):
                        i = i0 + l
                        off = jnp.where(v[l] >= H, DIM, 0)
                        for c in range(HC):
                            sl = pl.ds(c * LANES, LANES)
                            so = pl.ds(
                                pl.multiple_of(off + c * LANES, LANES), LANES)
                            acc_v[i, sl] = jnp.maximum(
                                acc_v[i, sl], _buf[i, so])
                    return c2

                lax.fori_loop(0, BPW // LANES, row16, 0)

                s_next = s_cur + NBUF

                @pl.when(s_next < SEQ)
                def _fire(_buf=bufs[k], _sem=sems[k], _s=s_next):
                    pltpu.make_async_copy(
                        tab2_hbm.at[gidx_v.at[_s]], _buf, _sem).start()

        return carry

    lax.fori_loop(0, (SEQ + NBUF - 1) // NBUF, group, 0)

    pltpu.sync_copy(acc_v, out_hbm.at[pl.ds(base, BPW), :])


def _linear_body(p_ref, w_ref, b_ref, o_ref):
    o_ref[...] = lax.dot_general(
        p_ref[...][:, :DIM], w_ref[...], (((1,), (1,)), ((), ())),
        preferred_element_type=jnp.float32) + b_ref[...]


def _trans_body(l_ref, r_ref, o_ref):
    x = jnp.concatenate([l_ref[...], r_ref[...]], axis=0)
    o_ref[...] = jnp.transpose(x, (1, 0))


def _widen_tc(tabT):
    grid = (H + TBLK - 1) // TBLK
    return pl.pallas_call(
        _trans_body,
        out_shape=jax.ShapeDtypeStruct((H, 2 * DIM), jnp.float32),
        grid=(grid,),
        in_specs=[
            pl.BlockSpec((DIM, TBLK), lambda i: (0, i)),
            pl.BlockSpec((DIM, TBLK), lambda i: (0, i + PAIR // TBLK)),
        ],
        out_specs=pl.BlockSpec((TBLK, 2 * DIM), lambda i: (i, 0)),
    )(tabT, tabT)


@jax.jit
def _run(x, table, W, b):
    mesh = plsc.VectorSubcoreMesh(core_axis_name="c", subcore_axis_name="s")
    params = pltpu.CompilerParams(use_tc_tiling_on_sc=True)
    pool = pl.kernel(
        _pool_body,
        out_type=jax.ShapeDtypeStruct((BATCH, 2 * DIM), jnp.float32),
        mesh=mesh,
        scratch_types=[
            pltpu.VMEM((SEQ, BPW), jnp.int32),
            pltpu.VMEM((SEQ, BPW), jnp.int32),
            pltpu.VMEM((BPW, 2 * DIM), jnp.float32),
        ] + [pltpu.VMEM((BPW, 2 * DIM), jnp.float32)] * NBUF
          + [pltpu.SemaphoreType.DMA] * NBUF,
        compiler_params=params,
    )
    tab2 = _widen_tc(table.T)
    pooled = pool(x, tab2)
    blk = 512
    return pl.pallas_call(
        _linear_body,
        out_shape=jax.ShapeDtypeStruct((BATCH, OUT_DIM), jnp.float32),
        grid=(BATCH // blk,),
        in_specs=[
            pl.BlockSpec((blk, 2 * DIM), lambda i: (i, 0)),
            pl.BlockSpec((OUT_DIM, DIM), lambda i: (0, 0)),
            pl.BlockSpec((1, OUT_DIM), lambda i: (0, 0)),
        ],
        out_specs=pl.BlockSpec((blk, OUT_DIM), lambda i: (i, 0)),
    )(pooled, W, b.reshape(1, OUT_DIM))


def kernel(x, table, W, b):
    return _run(x.astype(jnp.int32), table, W, b)


# speedup vs baseline: 2.1417x; 1.5464x over previous
import functools

import jax
import jax.numpy as jnp
from jax import lax
from jax.experimental import pallas as pl
from jax.experimental.pallas import tpu as pltpu
from jax.experimental.pallas import tpu_sc as plsc

SEQ = 200
BATCH = 4096
DIM = 64
OUT_DIM = 128
VOCAB = 1000000

NC = 2
NS = 16
NW = NC * NS
BPW = BATCH // NW
NBUF = 4
LANES = 16
HC = DIM // LANES

def _pool_body(x_hbm, tab2_hbm, out_hbm, idx_v, acc_v,
               b0, b1, b2, b3, s0, s1, s2, s3):
    bufs = (b0, b1, b2, b3)
    sems = (s0, s1, s2, s3)
    wid = lax.axis_index("s") * NC + lax.axis_index("c")
    base = pl.multiple_of(wid * BPW, BPW)

    pltpu.sync_copy(x_hbm.at[:, pl.ds(base, BPW)], idx_v)

    neg = jnp.full((LANES,), -jnp.inf, dtype=jnp.float32)

    def init_row(i, carry):
        for c in range(HC):
            acc_v[i, pl.ds(c * LANES, LANES)] = neg
        return carry

    lax.fori_loop(0, BPW, init_row, 0)

    for k in range(NBUF):
        pltpu.make_async_copy(
            tab2_hbm.at[idx_v.at[k]], bufs[k], sems[k]).start()

    def group(g, carry):
        for k in range(NBUF):
            s_cur = g * NBUF + k
            pltpu.make_async_copy(
                tab2_hbm.at[idx_v.at[s_cur]], bufs[k], sems[k]).wait()

            def row(i, c2, _buf=bufs[k]):
                for c in range(HC):
                    sl = pl.ds(c * LANES, LANES)
                    acc_v[i, sl] = jnp.maximum(acc_v[i, sl], _buf[i, sl])
                return c2

            lax.fori_loop(0, BPW, row, 0)

            s_next = s_cur + NBUF

            @pl.when(s_next < SEQ)
            def _fire(_buf=bufs[k], _sem=sems[k], _s=s_next):
                pltpu.make_async_copy(
                    tab2_hbm.at[idx_v.at[_s]], _buf, _sem).start()

        return carry

    lax.fori_loop(0, SEQ // NBUF, group, 0)

    pltpu.sync_copy(acc_v, out_hbm.at[pl.ds(base, BPW), :])


def _linear_body(p_ref, w_ref, b_ref, o_ref):
    o_ref[...] = lax.dot_general(
        p_ref[...][:, :DIM], w_ref[...], (((1,), (1,)), ((), ())),
        preferred_element_type=jnp.float32) + b_ref[...]


TBLK = 16384


def _trans_body(t_ref, o_ref):
    x = t_ref[...]
    o_ref[...] = jnp.transpose(jnp.concatenate([x, x], axis=0), (1, 0))


def _widen_tc(tabT):
    grid = (VOCAB + TBLK - 1) // TBLK
    return pl.pallas_call(
        _trans_body,
        out_shape=jax.ShapeDtypeStruct((VOCAB, 2 * DIM), jnp.float32),
        grid=(grid,),
        in_specs=[pl.BlockSpec((DIM, TBLK), lambda i: (0, i))],
        out_specs=pl.BlockSpec((TBLK, 2 * DIM), lambda i: (i, 0)),
    )(tabT)


@jax.jit
def _run(x, table, W, b):
    mesh = plsc.VectorSubcoreMesh(core_axis_name="c", subcore_axis_name="s")
    params = pltpu.CompilerParams(use_tc_tiling_on_sc=True)
    pool = pl.kernel(
        _pool_body,
        out_type=jax.ShapeDtypeStruct((BATCH, 2 * DIM), jnp.float32),
        mesh=mesh,
        scratch_types=[
            pltpu.VMEM((SEQ, BPW), jnp.int32),
            pltpu.VMEM((BPW, 2 * DIM), jnp.float32),
        ] + [pltpu.VMEM((BPW, 2 * DIM), jnp.float32)] * NBUF
          + [pltpu.SemaphoreType.DMA] * NBUF,
        compiler_params=params,
    )
    tab2 = _widen_tc(table.T)
    pooled = pool(x, tab2)
    blk = 512
    return pl.pallas_call(
        _linear_body,
        out_shape=jax.ShapeDtypeStruct((BATCH, OUT_DIM), jnp.float32),
        grid=(BATCH // blk,),
        in_specs=[
            pl.BlockSpec((blk, 2 * DIM), lambda i: (i, 0)),
            pl.BlockSpec((OUT_DIM, DIM), lambda i: (0, 0)),
            pl.BlockSpec((1, OUT_DIM), lambda i: (0, 0)),
        ],
        out_specs=pl.BlockSpec((blk, OUT_DIM), lambda i: (i, 0)),
    )(pooled, W, b.reshape(1, OUT_DIM))


def kernel(x, table, W, b):
    return _run(x.astype(jnp.int32), table, W, b)
